# baseline (speedup 1.0000x reference)
import functools
import math

import jax
import jax.numpy as jnp
from jax import lax
from jax.experimental import pallas as pl
from jax.experimental.pallas import tpu as pltpu
from jax.experimental.pallas import tpu_sc as plsc

D_MODEL = 64
DPAD = 128
BATCH = 16384
SEQ = 50
VOCAB = 1000000
NC, NS, L = 2, 16, 16
NW = NC * NS
BPT = BATCH // NW
KI = 128
BC = 256
GPC = BC // KI
NBLK = SEQ * (BPT // BC)
SCALE = math.sqrt(D_MODEL)

_mesh = plsc.VectorSubcoreMesh(core_axis_name="c", subcore_axis_name="s")


@functools.partial(
    pl.kernel,
    out_type=jax.ShapeDtypeStruct((SEQ, D_MODEL, BATCH), jnp.float32),
    mesh=_mesh,
    scratch_types=[
        [pltpu.VMEM((BC,), jnp.int32) for _ in range(2)],
        [pltpu.VMEM((BC, DPAD), jnp.float32) for _ in range(2)],
        [pltpu.VMEM((D_MODEL * BC,), jnp.float32) for _ in range(2)],
        pltpu.SemaphoreType.DMA,
        pltpu.SemaphoreType.DMA,
    ],
    compiler_params=pltpu.CompilerParams(
        use_tc_tiling_on_sc=True, needs_layout_passes=False
    ),
)
def _embed(xt_hbm, tab_hbm, out_hbm, idx_bufs, gath_bufs, stage_bufs, gsem, ssem):
    wid = lax.axis_index("s") * NC + lax.axis_index("c")
    base_b = wid * BPT

    def fire_gather(g, b):
        iv, gv = idx_bufs[b], gath_bufs[b]
        s = g // (BPT // BC)
        b0 = base_b + (g % (BPT // BC)) * BC
        pltpu.sync_copy(xt_hbm.at[s, pl.ds(b0, BC)], iv)
        for j in range(GPC):
            pltpu.async_copy(
                tab_hbm.at[iv.at[pl.ds(j * KI, KI)]],
                gv.at[pl.ds(j * KI, KI)],
                gsem,
            )

    def wait_gather(b):
        iv, gv = idx_bufs[b], gath_bufs[b]
        for j in range(GPC):
            pltpu.make_async_copy(
                tab_hbm.at[iv.at[pl.ds(j * KI, KI)]],
                gv.at[pl.ds(j * KI, KI)],
                gsem,
            ).wait()

    def fire_store(tv, s, b0):
        for d in range(D_MODEL):
            pltpu.async_copy(
                tv.at[pl.ds(d * BC, BC)],
                out_hbm.at[s, d, pl.ds(b0, BC)],
                ssem,
            )

    def drain_store(b):
        pltpu.make_async_copy(
            tab_hbm.at[pl.ds(0, 128)], gath_bufs[b].at[pl.ds(0, 128)], ssem
        ).wait()

    iota = lax.iota(jnp.int32, L)
    rots = [(iota + k) % 16 for k in range(16)]
    dvecs = [iota + c * L for c in range(D_MODEL // L)]
    svecs = [(iota + c * L) * BC for c in range(D_MODEL // L)]

    fire_gather(0, 0)
    fire_gather(1, 1)

    def step(g, carry):
        for b in range(2):
            gi = g * 2 + b
            gv, tv = gath_bufs[b], stage_bufs[b]
            wait_gather(b)

            @pl.when(gi >= 2)
            def _():
                drain_store(b)

            def tgroup(rg, cc):
                rbase = rg * 16
                for k in range(16):
                    rvec = rots[k] + rbase
                    for c in range(D_MODEL // L):
                        v = plsc.load_gather(gv, [rvec, dvecs[c]]) * SCALE
                        plsc.store_scatter(tv, [svecs[c] + rvec], v)
                return cc

            lax.fori_loop(0, BC // 16, tgroup, 0, unroll=4)

            s = gi // (BPT // BC)
            b0 = base_b + (gi % (BPT // BC)) * BC
            fire_store(tv, s, b0)

            @pl.when(gi + 2 < NBLK)
            def _():
                fire_gather(gi + 2, b)

        return carry

    lax.fori_loop(0, NBLK // 2, step, 0)
    drain_store(0)
    drain_store(1)


def kernel(x, table):
    xt = x.T
    tab128 = jnp.pad(table, ((0, 0), (0, DPAD - D_MODEL)))
    out = _embed(xt, tab128)
    return out.transpose(2, 0, 1)


# speedup vs baseline: 1.4984x; 1.4984x over previous
import functools
import math

import jax
import jax.numpy as jnp
from jax import lax
from jax.experimental import pallas as pl
from jax.experimental.pallas import tpu as pltpu
from jax.experimental.pallas import tpu_sc as plsc

D_MODEL = 64
DPAD = 128
BATCH = 16384
SEQ = 50
VOCAB = 1000000
NC, NS, L = 2, 16, 16
NW = NC * NS
BPT = BATCH // NW
KI = 128
BC = 256
GPC = BC // KI
NBLK = SEQ * (BPT // BC)
SCALE = math.sqrt(D_MODEL)

_mesh = plsc.VectorSubcoreMesh(core_axis_name="c", subcore_axis_name="s")


@functools.partial(
    pl.kernel,
    out_type=jax.ShapeDtypeStruct((SEQ, D_MODEL, BATCH), jnp.float32),
    mesh=_mesh,
    scratch_types=[
        [pltpu.VMEM((BC,), jnp.int32) for _ in range(2)],
        [pltpu.VMEM((BC, DPAD), jnp.float32) for _ in range(2)],
        [pltpu.VMEM((D_MODEL, BC), jnp.float32) for _ in range(2)],
        pltpu.SemaphoreType.DMA,
        pltpu.SemaphoreType.DMA,
    ],
    compiler_params=pltpu.CompilerParams(
        use_tc_tiling_on_sc=True, needs_layout_passes=False
    ),
)
def _embed(xt_hbm, tab_hbm, out_hbm, idx_bufs, gath_bufs, stage_bufs, gsem, ssem):
    wid = lax.axis_index("s") * NC + lax.axis_index("c")
    base_b = wid * BPT

    def fire_gather(g, b):
        iv, gv = idx_bufs[b], gath_bufs[b]
        s = g // (BPT // BC)
        b0 = base_b + (g % (BPT // BC)) * BC
        pltpu.sync_copy(xt_hbm.at[s, pl.ds(b0, BC)], iv)
        for j in range(GPC):
            pltpu.async_copy(
                tab_hbm.at[iv.at[pl.ds(j * KI, KI)]],
                gv.at[pl.ds(j * KI, KI)],
                gsem,
            )

    def wait_gather(b):
        iv, gv = idx_bufs[b], gath_bufs[b]
        for j in range(GPC):
            pltpu.make_async_copy(
                tab_hbm.at[iv.at[pl.ds(j * KI, KI)]],
                gv.at[pl.ds(j * KI, KI)],
                gsem,
            ).wait()

    def fire_store(tv, s, b0):
        pltpu.async_copy(tv, out_hbm.at[s, :, pl.ds(b0, BC)], ssem)

    def drain_store(b):
        pltpu.make_async_copy(
            stage_bufs[b], out_hbm.at[0, :, pl.ds(base_b, BC)], ssem
        ).wait()

    iota = lax.iota(jnp.int32, L)
    rots = [(iota + k) % 16 for k in range(16)]
    dvecs = [iota + c * L for c in range(D_MODEL // L)]

    fire_gather(0, 0)
    fire_gather(1, 1)

    def step(g, carry):
        for b in range(2):
            gi = g * 2 + b
            gv, tv = gath_bufs[b], stage_bufs[b]
            wait_gather(b)

            @pl.when(gi >= 2)
            def _():
                drain_store(b)

            def tgroup(rg, cc):
                rbase = rg * 16
                for k in range(16):
                    rvec = rots[k] + rbase
                    for c in range(D_MODEL // L):
                        v = plsc.load_gather(gv, [rvec, dvecs[c]]) * SCALE
                        plsc.store_scatter(tv, [dvecs[c], rvec], v)
                return cc

            lax.fori_loop(0, BC // 16, tgroup, 0)

            s = gi // (BPT // BC)
            b0 = base_b + (gi % (BPT // BC)) * BC
            fire_store(tv, s, b0)

            @pl.when(gi + 2 < NBLK)
            def _():
                fire_gather(gi + 2, b)

        return carry

    lax.fori_loop(0, NBLK // 2, step, 0)
    drain_store(0)
    drain_store(1)


def kernel(x, table):
    xt = x.T
    tab128 = jnp.pad(table, ((0, 0), (0, DPAD - D_MODEL)))
    out = _embed(xt, tab128)
    return out.transpose(2, 0, 1)


# speedup vs baseline: 1.5767x; 1.0522x over previous
import functools
import math

import jax
import jax.numpy as jnp
from jax import lax
from jax.experimental import pallas as pl
from jax.experimental.pallas import tpu as pltpu
from jax.experimental.pallas import tpu_sc as plsc

D_MODEL = 64
DPAD = 128
BATCH = 16384
SEQ = 50
VOCAB = 1000000
NC, NS, L = 2, 16, 16
NW = NC * NS
BPT = BATCH // NW
KI = 128
BC = 256
GPC = BC // KI
NBLK = SEQ * (BPT // BC)
SCALE = math.sqrt(D_MODEL)

_mesh = plsc.VectorSubcoreMesh(core_axis_name="c", subcore_axis_name="s")


@functools.partial(
    pl.kernel,
    out_type=jax.ShapeDtypeStruct((SEQ, D_MODEL, BATCH), jnp.float32),
    mesh=_mesh,
    scratch_types=[
        [pltpu.VMEM((BC,), jnp.int32) for _ in range(4)],
        [pltpu.VMEM((BC, DPAD), jnp.float32) for _ in range(2)],
        [pltpu.VMEM((D_MODEL, BC), jnp.float32) for _ in range(2)],
        pltpu.SemaphoreType.DMA,
        pltpu.SemaphoreType.DMA,
        pltpu.SemaphoreType.DMA,
    ],
    compiler_params=pltpu.CompilerParams(
        use_tc_tiling_on_sc=True, needs_layout_passes=False
    ),
)
def _embed(xt_hbm, tab_hbm, out_hbm, idx_bufs, gath_bufs, stage_bufs, gsem, ssem, isem):
    wid = lax.axis_index("s") * NC + lax.axis_index("c")
    base_b = wid * BPT

    def src_slice(g):
        s = g // (BPT // BC)
        b0 = base_b + (g % (BPT // BC)) * BC
        return xt_hbm.at[s, pl.ds(b0, BC)]

    def fire_idx(g, ib):
        pltpu.async_copy(src_slice(g), idx_bufs[ib], isem)

    def wait_idx(g, ib):
        pltpu.make_async_copy(src_slice(g), idx_bufs[ib], isem).wait()

    def fire_gather(ib, b):
        iv, gv = idx_bufs[ib], gath_bufs[b]
        for j in range(GPC):
            pltpu.async_copy(
                tab_hbm.at[iv.at[pl.ds(j * KI, KI)]],
                gv.at[pl.ds(j * KI, KI)],
                gsem,
            )

    def wait_gather(ib, b):
        iv, gv = idx_bufs[ib], gath_bufs[b]
        for j in range(GPC):
            pltpu.make_async_copy(
                tab_hbm.at[iv.at[pl.ds(j * KI, KI)]],
                gv.at[pl.ds(j * KI, KI)],
                gsem,
            ).wait()

    def fire_store(tv, s, b0):
        pltpu.async_copy(tv, out_hbm.at[s, :, pl.ds(b0, BC)], ssem)

    def drain_store(b):
        pltpu.make_async_copy(
            stage_bufs[b], out_hbm.at[0, :, pl.ds(base_b, BC)], ssem
        ).wait()

    iota = lax.iota(jnp.int32, L)
    rots = [(iota + k) % 16 for k in range(16)]
    dvecs = [iota + c * L for c in range(D_MODEL // L)]

    for g0 in range(4):
        fire_idx(g0, g0)
    wait_idx(0, 0)
    fire_gather(0, 0)
    wait_idx(1, 1)
    fire_gather(1, 1)

    def step(g, carry):
        for b in range(4):
            gi = g * 4 + b
            bg = b % 2
            gv, tv = gath_bufs[bg], stage_bufs[bg]
            wait_gather(b, bg)

            @pl.when(gi >= 2)
            def _():
                drain_store(bg)

            def tgroup(rg, cc):
                rbase = rg * 16
                for k in range(16):
                    rvec = rots[k] + rbase
                    for c in range(D_MODEL // L):
                        v = plsc.load_gather(gv, [rvec, dvecs[c]]) * SCALE
                        plsc.store_scatter(tv, [dvecs[c], rvec], v)
                return cc

            lax.fori_loop(0, BC // 16, tgroup, 0)

            s = gi // (BPT // BC)
            b0 = base_b + (gi % (BPT // BC)) * BC
            fire_store(tv, s, b0)

            @pl.when(gi + 2 < NBLK)
            def _():
                wait_idx(gi + 2, (b + 2) % 4)
                fire_gather((b + 2) % 4, bg)

            @pl.when(gi + 4 < NBLK)
            def _():
                fire_idx(gi + 4, b)

        return carry

    lax.fori_loop(0, NBLK // 4, step, 0)
    drain_store(0)
    drain_store(1)


def kernel(x, table):
    xt = x.T
    tab128 = jnp.pad(table, ((0, 0), (0, DPAD - D_MODEL)))
    out = _embed(xt, tab128)
    return out.transpose(2, 0, 1)
